# baseline (speedup 1.0000x reference)
import jax
import jax.numpy as jnp
from jax import lax
from jax.experimental import pallas as pl
from jax.experimental.pallas import tpu as pltpu
from jax.experimental.pallas import tpu_sc as plsc

NC = 2
NS = 16
NW = NC * NS
K = 128
ZB = 128


def _sc_degree(dst2d, cz, n_pad):
    cpt = dst2d.shape[0] // NW
    zr = n_pad // NS

    def body(dst_hbm, cz_hbm, out_hbm, didx, ones_v, dacc):
        cid = lax.axis_index("c")
        sid = lax.axis_index("s")
        wid = cid * NS + sid
        pltpu.sync_copy(cz_hbm.at[pl.ds(n_pad, K), :], ones_v)
        pltpu.sync_copy(dst_hbm.at[pl.ds(wid * cpt, cpt), :], didx)
        pltpu.sync_copy(cz_hbm.at[pl.ds(0, zr), :],
                        dacc.at[pl.ds(sid * zr, zr), :])
        plsc.subcore_barrier()

        def step(i, carry):
            pltpu.sync_copy(ones_v, dacc.at[didx.at[i]], add=True)
            return carry

        lax.fori_loop(0, cpt, step, 0)
        plsc.subcore_barrier()
        pltpu.sync_copy(dacc.at[pl.ds(sid * zr, zr), :],
                        out_hbm.at[cid, pl.ds(sid * zr, zr), :])

    fn = pl.kernel(
        body,
        out_type=jax.ShapeDtypeStruct((NC, n_pad, 8), jnp.float32),
        mesh=plsc.VectorSubcoreMesh(core_axis_name="c", subcore_axis_name="s"),
        scratch_types=[
            pltpu.VMEM((cpt, K), jnp.int32),
            pltpu.VMEM((K, 8), jnp.float32),
            pltpu.VMEM_SHARED((n_pad, 8), jnp.float32),
        ],
    )
    return fn(dst2d, cz)


def _sc_spmm(y, src, dst, n_pad):
    f = y.shape[1]
    e_pad = src.shape[0]
    ept = e_pad // NW
    cpt = ept // K
    zbr = 64

    zr = n_pad // NS

    def body(y_hbm, src_hbm, dst_hbm, out_hbm, sidx, didx, rows, zbuf, acc,
             gsem, isem):
        cid = lax.axis_index("c")
        sid = lax.axis_index("s")
        wid = cid * NS + sid
        base = wid * ept

        def zb(r, carry):
            for j in range(f // 16):
                zbuf[r, pl.ds(j * 16, 16)] = jnp.zeros((16,), jnp.float32)
            return carry

        lax.fori_loop(0, zbr, zb, 0)
        pltpu.sync_copy(src_hbm.at[pl.ds(base, K)], sidx.at[0])
        pltpu.sync_copy(dst_hbm.at[pl.ds(base, K)], didx.at[0])
        for k in range(zr // zbr):
            pltpu.sync_copy(zbuf, acc.at[pl.ds(sid * zr + k * zbr, zbr), :])
        plsc.subcore_barrier()

        pltpu.async_copy(y_hbm.at[sidx.at[0]], rows.at[0], gsem)
        pltpu.async_copy(src_hbm.at[pl.ds(base + K, K)], sidx.at[1], isem)
        pltpu.async_copy(dst_hbm.at[pl.ds(base + K, K)], didx.at[1], isem)

        def step(i, carry):
            b = i & 1
            nb = (i + 1) & 1
            pltpu.make_async_copy(y_hbm.at[sidx.at[b]], rows.at[b],
                                  gsem).wait()

            @pl.when(i + 1 < cpt)
            def _():
                pltpu.make_async_copy(src_hbm.at[pl.ds(0, K)], sidx.at[nb],
                                      isem).wait()
                pltpu.make_async_copy(dst_hbm.at[pl.ds(0, K)], didx.at[nb],
                                      isem).wait()
                pltpu.async_copy(y_hbm.at[sidx.at[nb]], rows.at[nb], gsem)

            pltpu.sync_copy(rows.at[b], acc.at[didx.at[b]], add=True)

            @pl.when(i + 2 < cpt)
            def _():
                off = base + (i + 2) * K
                pltpu.async_copy(src_hbm.at[pl.ds(off, K)], sidx.at[b], isem)
                pltpu.async_copy(dst_hbm.at[pl.ds(off, K)], didx.at[b], isem)

            return carry

        lax.fori_loop(0, cpt, step, 0)
        plsc.subcore_barrier()
        pltpu.sync_copy(acc.at[pl.ds(sid * zr, zr), :],
                        out_hbm.at[cid, pl.ds(sid * zr, zr), :])

    fn = pl.kernel(
        body,
        out_type=jax.ShapeDtypeStruct((NC, n_pad, f), jnp.float32),
        mesh=plsc.VectorSubcoreMesh(core_axis_name="c", subcore_axis_name="s"),
        scratch_types=[
            pltpu.VMEM((2, K), jnp.int32),
            pltpu.VMEM((2, K), jnp.int32),
            pltpu.VMEM((2, K, f), jnp.float32),
            pltpu.VMEM((zbr, f), jnp.float32),
            pltpu.VMEM_SHARED((n_pad, f), jnp.float32),
            pltpu.SemaphoreType.DMA,
            pltpu.SemaphoreType.DMA,
        ],
    )
    return fn(y, src, dst)


def _tc_pre(x, em, w0, b0, w1, degp):
    n = x.shape[0]

    def body(x_r, em_r, w0_r, b0_r, w1_r, deg_r, y1_r, dinv_r):
        nw0 = jnp.maximum(w0_r[...] @ em_r[...] + b0_r[...], 0.0)
        m1 = nw0 @ w1_r[...]
        dsum = deg_r[0, :n, 0:1] + deg_r[1, :n, 0:1] + 1.0
        dinv = lax.rsqrt(dsum)
        y1_r[...] = (x_r[...] @ m1) * dinv
        dinv_r[...] = dinv

    return pl.pallas_call(
        body,
        out_shape=[
            jax.ShapeDtypeStruct((n, x.shape[1]), jnp.float32),
            jax.ShapeDtypeStruct((n, 1), jnp.float32),
        ],
    )(x, em, w0, b0, w1, degp)


def _tc_mid(z, y, dinv, b, w_next):
    n, f = y.shape

    def body(z_r, y_r, dinv_r, b_r, w_r, out_r):
        ztot = z_r[0, :n] + z_r[1, :n] + y_r[...]
        h = jnp.maximum(ztot * dinv_r[...] + b_r[...], 0.0)
        out_r[...] = (h @ w_r[...]) * dinv_r[...]

    return pl.pallas_call(
        body,
        out_shape=jax.ShapeDtypeStruct((n, f), jnp.float32),
    )(z, y, dinv, b, w_next)


def _tc_post(z, y, dinv, b, ltw_t, ltb):
    n = y.shape[0]
    c = ltw_t.shape[1]

    def body(z_r, y_r, dinv_r, b_r, w_r, ltb_r, out_r):
        ztot = z_r[0, :n] + z_r[1, :n] + y_r[...]
        h = jnp.maximum(ztot * dinv_r[...] + b_r[...], 0.0)
        logits = h @ w_r[...] + ltb_r[...]
        m = jnp.max(logits, axis=1, keepdims=True)
        lse = jnp.log(jnp.sum(jnp.exp(logits - m), axis=1, keepdims=True)) + m
        out_r[...] = logits - lse

    return pl.pallas_call(
        body,
        out_shape=jax.ShapeDtypeStruct((n, c), jnp.float32),
    )(z, y, dinv, b, ltw_t, ltb)


def kernel(x, edge_index, E_meta, w0, b0, conv1_W, conv1_b, conv2_W, conv2_b,
           lt1_W, lt1_b):
    n, f = x.shape
    e = edge_index.shape[1]

    nblk = NS * ZB
    n_pad = ((n + 1 + nblk - 1) // nblk) * nblk
    cpt = ((e + NW * K - 1) // (NW * K) + 7) // 8 * 8
    e_pad = NW * cpt * K

    src = edge_index[0].astype(jnp.int32)
    dst = edge_index[1].astype(jnp.int32)
    if e_pad != e:
        src = jnp.concatenate([src, jnp.zeros((e_pad - e,), jnp.int32)])
        dst = jnp.concatenate([dst, jnp.full((e_pad - e,), n, jnp.int32)])
    src2d = src.reshape(NW * cpt, K)
    dst2d = dst.reshape(NW * cpt, K)

    cz = jnp.zeros((n_pad + K, 8), jnp.float32).at[n_pad:].set(1.0)

    degp = _sc_degree(dst2d, cz, n_pad)
    y1, dinv = _tc_pre(x, E_meta, w0, b0, conv1_W, degp)
    z1 = _sc_spmm(y1, src, dst, n_pad)
    y2 = _tc_mid(z1, y1, dinv, conv1_b.reshape(1, -1), conv2_W)
    z2 = _sc_spmm(y2, src, dst, n_pad)
    out = _tc_post(z2, y2, dinv, conv2_b.reshape(1, -1), lt1_W.T,
                   lt1_b.reshape(1, -1))
    return out


# speedup vs baseline: 3.0109x; 3.0109x over previous
import jax
import jax.numpy as jnp
from jax import lax
from jax.experimental import pallas as pl
from jax.experimental.pallas import tpu as pltpu
from jax.experimental.pallas import tpu_sc as plsc

NC = 2
NS = 16
NW = NC * NS
K = 128
ZB = 128


def _sc_degree(dst2d, cz, n_pad):
    cpt = dst2d.shape[0] // NW
    zr = n_pad // NS

    def body(dst_hbm, cz_hbm, out_hbm, didx, ones_v, dacc):
        cid = lax.axis_index("c")
        sid = lax.axis_index("s")
        wid = cid * NS + sid
        pltpu.sync_copy(cz_hbm.at[pl.ds(n_pad, K), :], ones_v)
        pltpu.sync_copy(dst_hbm.at[pl.ds(wid * cpt, cpt), :], didx)
        pltpu.sync_copy(cz_hbm.at[pl.ds(0, zr), :],
                        dacc.at[pl.ds(sid * zr, zr), :])
        plsc.subcore_barrier()

        def step(i, carry):
            pltpu.sync_copy(ones_v, dacc.at[didx.at[i]], add=True)
            return carry

        lax.fori_loop(0, cpt, step, 0)
        plsc.subcore_barrier()
        pltpu.sync_copy(dacc.at[pl.ds(sid * zr, zr), :],
                        out_hbm.at[cid, pl.ds(sid * zr, zr), :])

    fn = pl.kernel(
        body,
        out_type=jax.ShapeDtypeStruct((NC, n_pad, 8), jnp.float32),
        mesh=plsc.VectorSubcoreMesh(core_axis_name="c", subcore_axis_name="s"),
        scratch_types=[
            pltpu.VMEM((cpt, K), jnp.int32),
            pltpu.VMEM((K, 8), jnp.float32),
            pltpu.VMEM_SHARED((n_pad, 8), jnp.float32),
        ],
    )
    return fn(dst2d, cz)


def _sc_spmm(y, src, dst, n_pad):
    f = y.shape[1]
    e_pad = src.shape[0]
    ept = e_pad // NW
    cpt = ept // K
    zbr = 64

    zr = n_pad // NS

    def body(y_hbm, src_hbm, dst_hbm, out_hbm, sidx, didx, rows, zbuf, acc,
             gsem, isem):
        cid = lax.axis_index("c")
        sid = lax.axis_index("s")
        wid = cid * NS + sid
        base = wid * ept

        def zb(r, carry):
            for j in range(f // 16):
                zbuf[r, pl.ds(j * 16, 16)] = jnp.zeros((16,), jnp.float32)
            return carry

        lax.fori_loop(0, zbr, zb, 0)
        pltpu.sync_copy(src_hbm.at[pl.ds(base, K)], sidx.at[0])
        pltpu.sync_copy(dst_hbm.at[pl.ds(base, K)], didx.at[0])
        for k in range(zr // zbr):
            pltpu.sync_copy(zbuf, acc.at[pl.ds(sid * zr + k * zbr, zbr), :])
        plsc.subcore_barrier()

        pltpu.async_copy(y_hbm.at[sidx.at[0]], rows.at[0], gsem)
        pltpu.async_copy(src_hbm.at[pl.ds(base + K, K)], sidx.at[1], isem)
        pltpu.async_copy(dst_hbm.at[pl.ds(base + K, K)], didx.at[1], isem)

        def step(i, carry):
            b = i & 1
            nb = (i + 1) & 1
            pltpu.make_async_copy(y_hbm.at[sidx.at[b]], rows.at[b],
                                  gsem).wait()

            @pl.when(i + 1 < cpt)
            def _():
                pltpu.make_async_copy(src_hbm.at[pl.ds(0, K)], sidx.at[nb],
                                      isem).wait()
                pltpu.make_async_copy(dst_hbm.at[pl.ds(0, K)], didx.at[nb],
                                      isem).wait()
                pltpu.async_copy(y_hbm.at[sidx.at[nb]], rows.at[nb], gsem)

            pltpu.sync_copy(rows.at[b], acc.at[didx.at[b]], add=True)

            @pl.when(i + 2 < cpt)
            def _():
                off = base + (i + 2) * K
                pltpu.async_copy(src_hbm.at[pl.ds(off, K)], sidx.at[b], isem)
                pltpu.async_copy(dst_hbm.at[pl.ds(off, K)], didx.at[b], isem)

            return carry

        lax.fori_loop(0, cpt, step, 0)
        plsc.subcore_barrier()
        pltpu.sync_copy(acc.at[pl.ds(sid * zr, zr), :],
                        out_hbm.at[cid, pl.ds(sid * zr, zr), :])

    fn = pl.kernel(
        body,
        out_type=jax.ShapeDtypeStruct((NC, n_pad, f), jnp.float32),
        mesh=plsc.VectorSubcoreMesh(core_axis_name="c", subcore_axis_name="s"),
        scratch_types=[
            pltpu.VMEM((2, K), jnp.int32),
            pltpu.VMEM((2, K), jnp.int32),
            pltpu.VMEM((2, K, f), jnp.float32),
            pltpu.VMEM((zbr, f), jnp.float32),
            pltpu.VMEM_SHARED((n_pad, f), jnp.float32),
            pltpu.SemaphoreType.DMA,
            pltpu.SemaphoreType.DMA,
        ],
    )
    return fn(y, src, dst)


def _tc_pre(x, em, w0, b0, w1, degp):
    n = x.shape[0]

    def body(x_r, em_r, w0_r, b0_r, w1_r, deg_r, y1_r, dinv_r):
        nw0 = jnp.maximum(w0_r[...] @ em_r[...] + b0_r[...], 0.0)
        m1 = nw0 @ w1_r[...]
        dsum = deg_r[0, :n, 0:1] + deg_r[1, :n, 0:1] + 1.0
        dinv = lax.rsqrt(dsum)
        y1_r[...] = (x_r[...] @ m1) * dinv
        dinv_r[...] = dinv

    return pl.pallas_call(
        body,
        out_shape=[
            jax.ShapeDtypeStruct((n, x.shape[1]), jnp.float32),
            jax.ShapeDtypeStruct((n, 1), jnp.float32),
        ],
    )(x, em, w0, b0, w1, degp)


def _tc_mid(z, y, dinv, b, w_next):
    n, f = y.shape

    def body(z_r, y_r, dinv_r, b_r, w_r, out_r):
        ztot = z_r[0, :n] + z_r[1, :n] + y_r[...]
        h = jnp.maximum(ztot * dinv_r[...] + b_r[...], 0.0)
        out_r[...] = (h @ w_r[...]) * dinv_r[...]

    return pl.pallas_call(
        body,
        out_shape=jax.ShapeDtypeStruct((n, f), jnp.float32),
    )(z, y, dinv, b, w_next)


def _tc_post(z, y, dinv, b, ltw_t, ltb):
    n = y.shape[0]
    c = ltw_t.shape[1]

    def body(z_r, y_r, dinv_r, b_r, w_r, ltb_r, out_r):
        ztot = z_r[0, :n] + z_r[1, :n] + y_r[...]
        h = jnp.maximum(ztot * dinv_r[...] + b_r[...], 0.0)
        logits = h @ w_r[...] + ltb_r[...]
        m = jnp.max(logits, axis=1, keepdims=True)
        lse = jnp.log(jnp.sum(jnp.exp(logits - m), axis=1, keepdims=True)) + m
        out_r[...] = logits - lse

    return pl.pallas_call(
        body,
        out_shape=jax.ShapeDtypeStruct((n, c), jnp.float32),
    )(z, y, dinv, b, ltw_t, ltb)


def kernel(x, edge_index, E_meta, w0, b0, conv1_W, conv1_b, conv2_W, conv2_b,
           lt1_W, lt1_b):
    n, f = x.shape
    e = edge_index.shape[1]

    nblk = NS * ZB
    n_pad = ((n + 1 + nblk - 1) // nblk) * nblk
    cpt = ((e + NW * K - 1) // (NW * K) + 7) // 8 * 8
    e_pad = NW * cpt * K

    src = edge_index[0].astype(jnp.int32)
    dst = edge_index[1].astype(jnp.int32)
    if e_pad != e:
        pad = e_pad - e
        ar = jnp.arange(pad, dtype=jnp.int32)
        src = jnp.concatenate([src, ar % n])
        dst = jnp.concatenate([dst, n + ar % (n_pad - n)])
    src2d = src.reshape(NW * cpt, K)
    dst2d = dst.reshape(NW * cpt, K)

    cz = jnp.zeros((n_pad + K, 8), jnp.float32).at[n_pad:].set(1.0)

    degp = _sc_degree(dst2d, cz, n_pad)
    y1, dinv = _tc_pre(x, E_meta, w0, b0, conv1_W, degp)
    z1 = _sc_spmm(y1, src, dst, n_pad)
    y2 = _tc_mid(z1, y1, dinv, conv1_b.reshape(1, -1), conv2_W)
    z2 = _sc_spmm(y2, src, dst, n_pad)
    out = _tc_post(z2, y2, dinv, conv2_b.reshape(1, -1), lt1_W.T,
                   lt1_b.reshape(1, -1))
    return out
